# baseline (speedup 1.0000x reference)
import functools

import jax
import jax.numpy as jnp
from jax import lax
from jax.experimental import pallas as pl
from jax.experimental.pallas import tpu as pltpu
from jax.experimental.pallas import tpu_sc as plsc

N = 50000
E = 800000
G = 128
H = 64
HH = 32

NSUB = 16
EPAD = 819200
EPC = EPAD // NSUB
CH = 128
NCH = EPC // CH

ROWS_A = 3128
ROWS_B = 3080
ZR = 280

R = 2000
GRID = N // R


def _sc_mesh():
    return plsc.VectorSubcoreMesh(core_axis_name="c", subcore_axis_name="s")


@functools.partial(
    pl.kernel,
    mesh=_sc_mesh(),
    compiler_params=pltpu.CompilerParams(use_tc_tiling_on_sc=False, needs_layout_passes=False),
    out_type=jax.ShapeDtypeStruct((2, N, HH), jnp.float32),
    scratch_types=[
        pltpu.VMEM((8, 3, CH), jnp.int32),
        pltpu.VMEM((4, CH, HH), jnp.float32),
        pltpu.VMEM((ZR, HH), jnp.float32),
        pltpu.VMEM_SHARED((N, HH), jnp.float32),
        pltpu.SemaphoreType.DMA, pltpu.SemaphoreType.DMA,
        pltpu.SemaphoreType.DMA, pltpu.SemaphoreType.DMA,
        pltpu.SemaphoreType.DMA, pltpu.SemaphoreType.DMA,
        pltpu.SemaphoreType.DMA, pltpu.SemaphoreType.DMA,
        pltpu.SemaphoreType.DMA, pltpu.SemaphoreType.DMA,
        pltpu.SemaphoreType.DMA, pltpu.SemaphoreType.DMA,
        pltpu.SemaphoreType.DMA, pltpu.SemaphoreType.DMA,
        pltpu.SemaphoreType.DMA, pltpu.SemaphoreType.DMA,
    ],
)
def _edge_kernel(asp, ep_h, out_h, ebuf, rows, zbuf, acc,
                 es0, es1, es2, es3, es4, es5, es6, es7,
                 gs0, gs1, gs2, gs3, ss0, ss1, ss2, ss3):
    cid = lax.axis_index("c")
    sid = lax.axis_index("s")
    esems = [es0, es1, es2, es3, es4, es5, es6, es7]
    gsems = [gs0, gs1, gs2, gs3]
    ssems = [ss0, ss1, ss2, ss3]

    zvec = jnp.zeros((16,), jnp.float32)

    def zfill(r, _):
        zbuf[r, pl.ds(0, 16)] = zvec
        zbuf[r, pl.ds(16, 16)] = zvec
        return 0
    lax.fori_loop(0, ZR, zfill, 0)

    rbase = sid * ROWS_A

    def zcopy(j, _):
        pltpu.sync_copy(zbuf, acc.at[pl.ds(rbase + j * ZR, ZR)])
        return 0
    lax.fori_loop(0, ROWS_B // ZR, zcopy, 0)

    @pl.when(sid < NSUB - 1)
    def _():
        pltpu.sync_copy(zbuf.at[pl.ds(0, ROWS_A - ROWS_B)],
                        acc.at[pl.ds(rbase + ROWS_B, ROWS_A - ROWS_B)])

    plsc.subcore_barrier()

    gch0 = sid * NCH

    def e_issue(gj, eb):
        pltpu.async_copy(ep_h.at[gj], ebuf.at[eb], esems[eb])

    def e_wait(gj, eb):
        pltpu.make_async_copy(ep_h.at[gj], ebuf.at[eb], esems[eb]).wait()

    def g_issue(eb, b):
        pltpu.async_copy(asp.at[cid].at[ebuf.at[eb].at[0]], rows.at[b],
                         gsems[b])

    def g_wait(eb, b):
        pltpu.make_async_copy(asp.at[cid].at[ebuf.at[eb].at[0]], rows.at[b],
                              gsems[b]).wait()

    def s_issue(eb, b):
        pltpu.async_copy(rows.at[b], acc.at[ebuf.at[eb].at[1]], ssems[b],
                         add=True)

    def s_wait(eb, b):
        pltpu.make_async_copy(rows.at[b], acc.at[ebuf.at[eb].at[1]],
                              ssems[b]).wait()

    def scale(eb, b):
        def body(g, _):
            wi = ebuf[eb, 2, pl.ds(g * 16, 16)]
            w = plsc.bitcast(wi, jnp.float32)
            base = g * 16
            for l in range(16):
                s = w[l]
                e = base + l
                rows[b, e, pl.ds(0, 16)] = rows[b, e, pl.ds(0, 16)] * s
                rows[b, e, pl.ds(16, 16)] = rows[b, e, pl.ds(16, 16)] * s
            return 0
        lax.fori_loop(0, CH // 16, body, 0)

    for eb in range(6):
        e_issue(gch0 + eb, eb)
    e_wait(gch0, 0)
    g_issue(0, 0)
    e_wait(gch0 + 1, 1)
    g_issue(1, 1)

    NOUT = NCH // 8

    def outer(jj, _):
        for bb in range(8):
            j = 8 * jj + bb
            b = bb % 4
            eb = bb % 8
            b2 = (bb + 2) % 4
            eb2p = (bb + 6) % 8
            eb2n = (bb + 2) % 8

            g_wait(eb, b)

            if bb >= 2:
                s_wait(eb2p, b2)
            else:
                @pl.when(jj >= 1)
                def _():
                    s_wait(eb2p, b2)

            if bb < 2:
                e_issue(gch0 + j + 6, eb2p)
            else:
                @pl.when(jj < NOUT - 1)
                def _():
                    e_issue(gch0 + j + 6, eb2p)

            if bb < 6:
                e_wait(gch0 + j + 2, eb2n)
                g_issue(eb2n, b2)
            else:
                @pl.when(jj < NOUT - 1)
                def _():
                    e_wait(gch0 + j + 2, eb2n)
                    g_issue(eb2n, b2)

            s_issue(eb, b)
        return 0
    lax.fori_loop(0, NOUT, outer, 0)

    s_wait(6, 2)
    s_wait(7, 3)

    plsc.subcore_barrier()

    pltpu.sync_copy(acc.at[pl.ds(rbase, ROWS_B)],
                    out_h.at[cid].at[pl.ds(rbase, ROWS_B)])

    @pl.when(sid < NSUB - 1)
    def _():
        pltpu.sync_copy(acc.at[pl.ds(rbase + ROWS_B, ROWS_A - ROWS_B)],
                        out_h.at[cid].at[pl.ds(rbase + ROWS_B, ROWS_A - ROWS_B)])


NW = 32
EPT = EPAD // NW
NCHW = EPT // CH


@functools.partial(
    pl.kernel,
    mesh=_sc_mesh(),
    compiler_params=pltpu.CompilerParams(use_tc_tiling_on_sc=False, needs_layout_passes=False),
    out_type=jax.ShapeDtypeStruct((NW, N), jnp.float32),
    scratch_types=[
        pltpu.VMEM((8, 3, CH), jnp.int32),
        pltpu.VMEM((N,), jnp.float32),
        pltpu.SemaphoreType.DMA, pltpu.SemaphoreType.DMA,
        pltpu.SemaphoreType.DMA, pltpu.SemaphoreType.DMA,
        pltpu.SemaphoreType.DMA, pltpu.SemaphoreType.DMA,
        pltpu.SemaphoreType.DMA, pltpu.SemaphoreType.DMA,
    ],
)
def _wdeg_kernel(ep_h, out_h, ebuf, accv,
                 es0, es1, es2, es3, es4, es5, es6, es7):
    cid = lax.axis_index("c")
    sid = lax.axis_index("s")
    wid = sid * 2 + cid
    esems = [es0, es1, es2, es3, es4, es5, es6, es7]

    zvec = jnp.zeros((16,), jnp.float32)

    def zfill(r, _):
        accv[pl.ds(r * 16, 16)] = zvec
        return 0
    lax.fori_loop(0, N // 16, zfill, 0)

    gch0 = wid * NCHW

    def e_issue(gj, eb):
        pltpu.async_copy(ep_h.at[gj], ebuf.at[eb], esems[eb])

    def e_wait(gj, eb):
        pltpu.make_async_copy(ep_h.at[gj], ebuf.at[eb], esems[eb]).wait()

    for eb in range(6):
        e_issue(gch0 + eb, eb)

    def outer(jj, _):
        for bb in range(8):
            j = 8 * jj + bb
            e_wait(gch0 + j, bb)

            if bb < 2:
                e_issue(gch0 + j + 6, (bb + 6) % 8)
            else:
                @pl.when(jj < NCHW // 8 - 1)
                def _():
                    e_issue(gch0 + j + 6, (bb + 6) % 8)

            def fill(g, _):
                idx = ebuf[bb, 1, pl.ds(g * 16, 16)]
                w = plsc.bitcast(ebuf[bb, 2, pl.ds(g * 16, 16)], jnp.float32)
                plsc.addupdate_scatter(accv, [idx], w)
                return 0
            lax.fori_loop(0, CH // 16, fill, 0)
        return 0
    lax.fori_loop(0, NCHW // 8, outer, 0)

    pltpu.sync_copy(accv, out_h.at[wid])


def _full(shape):
    return pl.BlockSpec(shape, lambda i: tuple(0 for _ in shape))


def _wreduce_body(wp_ref, wdeg_ref):
    wdeg_ref[...] = jnp.sum(wp_ref[...], axis=0)[:, None]


def _wreduce(wparts):
    return pl.pallas_call(
        _wreduce_body,
        grid=(1,),
        in_specs=[_full((NW, N))],
        out_specs=[pl.BlockSpec((N, 1), lambda i: (0, 0))],
        out_shape=[jax.ShapeDtypeStruct((N, 1), jnp.float32)],
    )(wparts)[0]


def _prep_body(x_ref, nwt, nb, w1t, b1, w2t, w3t, b3,
               asp_ref, b_ref, c_ref):
    h = jnp.dot(x_ref[...], nwt[...], preferred_element_type=jnp.float32) + nb[...]
    a = jnp.dot(h, w1t[...], preferred_element_type=jnp.float32) + b1[...]
    asp_ref[0] = a[:, :HH]
    asp_ref[1] = a[:, HH:]
    b_ref[...] = jnp.dot(h, w2t[...], preferred_element_type=jnp.float32)
    c_ref[...] = jnp.dot(h, w3t[...], preferred_element_type=jnp.float32) + b3[...]


def _combine_body(agg_ref, bm_ref, cm_ref, wdeg_ref, w1t, b1, w2t, w3t, b3,
                  asp_ref, b_ref, c_ref):
    agg = jnp.concatenate([agg_ref[0], agg_ref[1]], axis=1)
    h = jnp.maximum(agg - bm_ref[...] * wdeg_ref[...] + cm_ref[...], 0.0)
    a = jnp.dot(h, w1t[...], preferred_element_type=jnp.float32) + b1[...]
    asp_ref[0] = a[:, :HH]
    asp_ref[1] = a[:, HH:]
    b_ref[...] = jnp.dot(h, w2t[...], preferred_element_type=jnp.float32)
    c_ref[...] = jnp.dot(h, w3t[...], preferred_element_type=jnp.float32) + b3[...]


def _pool_body(agg_ref, bm_ref, cm_ref, wdeg_ref, batch_ref,
               l1t, l1b, l2t, l2b, pred_ref, sums, counts):
    i = pl.program_id(0)
    agg = jnp.concatenate([agg_ref[0], agg_ref[1]], axis=1)
    h = jnp.maximum(agg - bm_ref[...] * wdeg_ref[...] + cm_ref[...], 0.0)
    bidx = batch_ref[...][:, 0]
    oh = (bidx[:, None] == lax.broadcasted_iota(jnp.int32, (R, G), 1)
          ).astype(jnp.float32)

    @pl.when(i == 0)
    def _():
        sums[...] = jnp.zeros((G, H), jnp.float32)
        counts[...] = jnp.zeros((G, 1), jnp.float32)

    sums[...] += lax.dot_general(oh, h, (((0,), (0,)), ((), ())),
                                 preferred_element_type=jnp.float32)
    counts[...] += jnp.sum(oh, axis=0)[:, None]

    @pl.when(i == GRID - 1)
    def _():
        gx = sums[...] / jnp.maximum(counts[...], 1.0)
        p = jnp.maximum(
            jnp.dot(gx, l1t[...], preferred_element_type=jnp.float32) + l1b[...],
            0.0)
        pred_ref[...] = jnp.dot(p, l2t[...],
                                preferred_element_type=jnp.float32) + l2b[...]


def _row_spec(cols):
    return pl.BlockSpec((R, cols), lambda i: (i, 0))


def _split_spec(cols):
    return pl.BlockSpec((2, R, cols), lambda i: (0, i, 0))


def _prep(x8, nwt, nb, w1t, b1, w2t, w3t, b3):
    return pl.pallas_call(
        _prep_body,
        grid=(GRID,),
        in_specs=[
            _row_spec(8), _full((8, H)), _full((1, H)),
            _full((H, H)), _full((1, H)), _full((H, H)),
            _full((H, H)), _full((1, H)),
        ],
        out_specs=[_split_spec(HH), _row_spec(H), _row_spec(H)],
        out_shape=[
            jax.ShapeDtypeStruct((2, N, HH), jnp.float32),
            jax.ShapeDtypeStruct((N, H), jnp.float32),
            jax.ShapeDtypeStruct((N, H), jnp.float32),
        ],
    )(x8, nwt, nb, w1t, b1, w2t, w3t, b3)


def _combine(agg, bm, cm, wdeg, w1t, b1, w2t, w3t, b3):
    return pl.pallas_call(
        _combine_body,
        grid=(GRID,),
        in_specs=[
            _split_spec(HH), _row_spec(H), _row_spec(H), _row_spec(1),
            _full((H, H)), _full((1, H)), _full((H, H)),
            _full((H, H)), _full((1, H)),
        ],
        out_specs=[_split_spec(HH), _row_spec(H), _row_spec(H)],
        out_shape=[
            jax.ShapeDtypeStruct((2, N, HH), jnp.float32),
            jax.ShapeDtypeStruct((N, H), jnp.float32),
            jax.ShapeDtypeStruct((N, H), jnp.float32),
        ],
    )(agg, bm, cm, wdeg, w1t, b1, w2t, w3t, b3)


def _pool(agg, bm, cm, wdeg, batch2, l1t, l1b, l2t, l2b):
    return pl.pallas_call(
        _pool_body,
        grid=(GRID,),
        in_specs=[
            _split_spec(HH), _row_spec(H), _row_spec(H), _row_spec(1),
            _row_spec(1),
            _full((H, G)), _full((1, G)), _full((G, 3)), _full((1, 3)),
        ],
        out_specs=[pl.BlockSpec((G, 3), lambda i: (0, 0))],
        out_shape=[jax.ShapeDtypeStruct((G, 3), jnp.float32)],
        scratch_shapes=[
            pltpu.VMEM((G, H), jnp.float32),
            pltpu.VMEM((G, 1), jnp.float32),
        ],
    )(agg, bm, cm, wdeg, batch2, l1t, l1b, l2t, l2b)[0]


def kernel(x, edge_index, edge_attr, batch, node_W, node_b,
           W1s, b1s, W2s, W3s, b3s, lin1_W, lin1_b, lin2_W, lin2_b):
    pad = EPAD - E
    src = jnp.concatenate([edge_index[0], jnp.zeros((pad,), jnp.int32)])
    dst = jnp.concatenate([edge_index[1], jnp.zeros((pad,), jnp.int32)])
    ew = jnp.concatenate([edge_attr, jnp.zeros((pad,), jnp.float32)])
    epack = jnp.concatenate(
        [src.reshape(-1, 1, CH), dst.reshape(-1, 1, CH),
         lax.bitcast_convert_type(ew, jnp.int32).reshape(-1, 1, CH)],
        axis=1)

    x8 = jnp.concatenate([x, jnp.zeros((N, 3), jnp.float32)], axis=1)
    nwt = jnp.concatenate([node_W.T, jnp.zeros((3, H), jnp.float32)], axis=0)

    wparts = _wdeg_kernel(epack)
    wdeg = _wreduce(wparts)

    asp, bm, cm = _prep(
        x8, nwt, node_b[None, :],
        W1s[0].T, b1s[0][None, :], W2s[0].T, W3s[0].T, b3s[0][None, :])

    for i in (1, 2):
        agg = _edge_kernel(asp, epack)
        asp, bm, cm = _combine(
            agg, bm, cm, wdeg,
            W1s[i].T, b1s[i][None, :], W2s[i].T, W3s[i].T, b3s[i][None, :])

    agg = _edge_kernel(asp, epack)
    return _pool(agg, bm, cm, wdeg, batch[:, None],
                 lin1_W.T, lin1_b[None, :], lin2_W.T, lin2_b[None, :])


# speedup vs baseline: 1.0046x; 1.0046x over previous
import functools

import jax
import jax.numpy as jnp
from jax import lax
from jax.experimental import pallas as pl
from jax.experimental.pallas import tpu as pltpu
from jax.experimental.pallas import tpu_sc as plsc

N = 50000
E = 800000
G = 128
H = 64
HH = 32

NSUB = 16
EPAD = 819200
EPC = EPAD // NSUB
CH = 128
NCH = EPC // CH

ROWS_A = 3128
ROWS_B = 3080
ZR = 280

R = 2000
GRID = N // R


def _sc_mesh():
    return plsc.VectorSubcoreMesh(core_axis_name="c", subcore_axis_name="s")


@functools.partial(
    pl.kernel,
    mesh=_sc_mesh(),
    compiler_params=pltpu.CompilerParams(use_tc_tiling_on_sc=False, needs_layout_passes=False),
    out_type=jax.ShapeDtypeStruct((2, N, HH), jnp.float32),
    scratch_types=[
        pltpu.VMEM((10, 3, CH), jnp.int32),
        pltpu.VMEM((5, CH, HH), jnp.float32),
        pltpu.VMEM_SHARED((N, HH), jnp.float32),
        pltpu.SemaphoreType.DMA, pltpu.SemaphoreType.DMA,
        pltpu.SemaphoreType.DMA, pltpu.SemaphoreType.DMA,
        pltpu.SemaphoreType.DMA, pltpu.SemaphoreType.DMA,
        pltpu.SemaphoreType.DMA, pltpu.SemaphoreType.DMA,
        pltpu.SemaphoreType.DMA, pltpu.SemaphoreType.DMA,
        pltpu.SemaphoreType.DMA, pltpu.SemaphoreType.DMA,
        pltpu.SemaphoreType.DMA, pltpu.SemaphoreType.DMA,
        pltpu.SemaphoreType.DMA, pltpu.SemaphoreType.DMA,
        pltpu.SemaphoreType.DMA, pltpu.SemaphoreType.DMA,
        pltpu.SemaphoreType.DMA, pltpu.SemaphoreType.DMA,
    ],
)
def _edge_kernel(asp, ep_h, zz_h, out_h, ebuf, rows, acc,
                 es0, es1, es2, es3, es4, es5, es6, es7, es8, es9,
                 gs0, gs1, gs2, gs3, gs4, ss0, ss1, ss2, ss3, ss4):
    cid = lax.axis_index("c")
    sid = lax.axis_index("s")
    esems = [es0, es1, es2, es3, es4, es5, es6, es7, es8, es9]
    gsems = [gs0, gs1, gs2, gs3, gs4]
    ssems = [ss0, ss1, ss2, ss3, ss4]

    rbase = sid * ROWS_A

    def zcopy(j, _):
        pltpu.sync_copy(zz_h, acc.at[pl.ds(rbase + j * ZR, ZR)])
        return 0
    lax.fori_loop(0, ROWS_B // ZR, zcopy, 0)

    @pl.when(sid < NSUB - 1)
    def _():
        pltpu.sync_copy(zz_h.at[pl.ds(0, ROWS_A - ROWS_B)],
                        acc.at[pl.ds(rbase + ROWS_B, ROWS_A - ROWS_B)])

    plsc.subcore_barrier()

    gch0 = sid * NCH

    def e_issue(gj, eb):
        pltpu.async_copy(ep_h.at[gj], ebuf.at[eb], esems[eb])

    def e_wait(gj, eb):
        pltpu.make_async_copy(ep_h.at[gj], ebuf.at[eb], esems[eb]).wait()

    def g_issue(eb, b):
        pltpu.async_copy(asp.at[cid].at[ebuf.at[eb].at[0]], rows.at[b],
                         gsems[b])

    def g_wait(eb, b):
        pltpu.make_async_copy(asp.at[cid].at[ebuf.at[eb].at[0]], rows.at[b],
                              gsems[b]).wait()

    def s_issue(eb, b):
        pltpu.async_copy(rows.at[b], acc.at[ebuf.at[eb].at[1]], ssems[b],
                         add=True)

    def s_wait(eb, b):
        pltpu.make_async_copy(rows.at[b], acc.at[ebuf.at[eb].at[1]],
                              ssems[b]).wait()

    def scale(eb, b):
        def body(g, _):
            wi = ebuf[eb, 2, pl.ds(g * 16, 16)]
            w = plsc.bitcast(wi, jnp.float32)
            base = g * 16
            for l in range(16):
                s = w[l]
                e = base + l
                rows[b, e, pl.ds(0, 16)] = rows[b, e, pl.ds(0, 16)] * s
                rows[b, e, pl.ds(16, 16)] = rows[b, e, pl.ds(16, 16)] * s
            return 0
        lax.fori_loop(0, CH // 16, body, 0)

    for eb in range(6):
        e_issue(gch0 + eb, eb)
    for k in range(3):
        e_wait(gch0 + k, k)
        g_issue(k, k)

    NOUT = NCH // 10

    def outer(jj, _):
        for bb in range(10):
            j = 10 * jj + bb
            b = bb % 5
            bp = (bb + 3) % 5
            ebp = (bb + 8) % 10
            eb6 = (bb + 6) % 10
            eb3 = (bb + 3) % 10

            g_wait(bb, b)

            if bb >= 2:
                s_wait(ebp, bp)
            else:
                @pl.when(jj >= 1)
                def _():
                    s_wait(ebp, bp)

            if bb < 4:
                e_issue(gch0 + j + 6, eb6)
            else:
                @pl.when(jj < NOUT - 1)
                def _():
                    e_issue(gch0 + j + 6, eb6)

            if bb < 7:
                e_wait(gch0 + j + 3, eb3)
                g_issue(eb3, bp)
            else:
                @pl.when(jj < NOUT - 1)
                def _():
                    e_wait(gch0 + j + 3, eb3)
                    g_issue(eb3, bp)

            scale(bb, b)
            s_issue(bb, b)
        return 0
    lax.fori_loop(0, NOUT, outer, 0)

    s_wait(8, 3)
    s_wait(9, 4)

    plsc.subcore_barrier()

    pltpu.sync_copy(acc.at[pl.ds(rbase, ROWS_B)],
                    out_h.at[cid].at[pl.ds(rbase, ROWS_B)])

    @pl.when(sid < NSUB - 1)
    def _():
        pltpu.sync_copy(acc.at[pl.ds(rbase + ROWS_B, ROWS_A - ROWS_B)],
                        out_h.at[cid].at[pl.ds(rbase + ROWS_B, ROWS_A - ROWS_B)])


NW = 32
EPT = EPAD // NW
NCHW = EPT // CH


@functools.partial(
    pl.kernel,
    mesh=_sc_mesh(),
    compiler_params=pltpu.CompilerParams(use_tc_tiling_on_sc=False, needs_layout_passes=False),
    out_type=jax.ShapeDtypeStruct((NW, N), jnp.float32),
    scratch_types=[
        pltpu.VMEM((8, 3, CH), jnp.int32),
        pltpu.VMEM((N,), jnp.float32),
        pltpu.SemaphoreType.DMA, pltpu.SemaphoreType.DMA,
        pltpu.SemaphoreType.DMA, pltpu.SemaphoreType.DMA,
        pltpu.SemaphoreType.DMA, pltpu.SemaphoreType.DMA,
        pltpu.SemaphoreType.DMA, pltpu.SemaphoreType.DMA,
    ],
)
def _wdeg_kernel(ep_h, out_h, ebuf, accv,
                 es0, es1, es2, es3, es4, es5, es6, es7):
    cid = lax.axis_index("c")
    sid = lax.axis_index("s")
    wid = sid * 2 + cid
    esems = [es0, es1, es2, es3, es4, es5, es6, es7]

    zvec = jnp.zeros((16,), jnp.float32)

    def zfill(r, _):
        accv[pl.ds(r * 16, 16)] = zvec
        return 0
    lax.fori_loop(0, N // 16, zfill, 0)

    gch0 = wid * NCHW

    def e_issue(gj, eb):
        pltpu.async_copy(ep_h.at[gj], ebuf.at[eb], esems[eb])

    def e_wait(gj, eb):
        pltpu.make_async_copy(ep_h.at[gj], ebuf.at[eb], esems[eb]).wait()

    for eb in range(6):
        e_issue(gch0 + eb, eb)

    def outer(jj, _):
        for bb in range(8):
            j = 8 * jj + bb
            e_wait(gch0 + j, bb)

            if bb < 2:
                e_issue(gch0 + j + 6, (bb + 6) % 8)
            else:
                @pl.when(jj < NCHW // 8 - 1)
                def _():
                    e_issue(gch0 + j + 6, (bb + 6) % 8)

            def fill(g, _):
                idx = ebuf[bb, 1, pl.ds(g * 16, 16)]
                w = plsc.bitcast(ebuf[bb, 2, pl.ds(g * 16, 16)], jnp.float32)
                plsc.addupdate_scatter(accv, [idx], w)
                return 0
            lax.fori_loop(0, CH // 16, fill, 0)
        return 0
    lax.fori_loop(0, NCHW // 8, outer, 0)

    pltpu.sync_copy(accv, out_h.at[wid])


def _full(shape):
    return pl.BlockSpec(shape, lambda i: tuple(0 for _ in shape))


def _wreduce_body(wp_ref, wdeg_ref):
    wdeg_ref[...] = jnp.sum(wp_ref[...], axis=0)[:, None]


def _wreduce(wparts):
    return pl.pallas_call(
        _wreduce_body,
        grid=(1,),
        in_specs=[_full((NW, N))],
        out_specs=[pl.BlockSpec((N, 1), lambda i: (0, 0))],
        out_shape=[jax.ShapeDtypeStruct((N, 1), jnp.float32)],
    )(wparts)[0]


def _prep_body(x_ref, nwt, nb, w1t, b1, w2t, w3t, b3,
               asp_ref, b_ref, c_ref):
    h = jnp.dot(x_ref[...], nwt[...], preferred_element_type=jnp.float32) + nb[...]
    a = jnp.dot(h, w1t[...], preferred_element_type=jnp.float32) + b1[...]
    asp_ref[0] = a[:, :HH]
    asp_ref[1] = a[:, HH:]
    b_ref[...] = jnp.dot(h, w2t[...], preferred_element_type=jnp.float32)
    c_ref[...] = jnp.dot(h, w3t[...], preferred_element_type=jnp.float32) + b3[...]


def _combine_body(agg_ref, bm_ref, cm_ref, wdeg_ref, w1t, b1, w2t, w3t, b3,
                  asp_ref, b_ref, c_ref):
    agg = jnp.concatenate([agg_ref[0], agg_ref[1]], axis=1)
    h = jnp.maximum(agg - bm_ref[...] * wdeg_ref[...] + cm_ref[...], 0.0)
    a = jnp.dot(h, w1t[...], preferred_element_type=jnp.float32) + b1[...]
    asp_ref[0] = a[:, :HH]
    asp_ref[1] = a[:, HH:]
    b_ref[...] = jnp.dot(h, w2t[...], preferred_element_type=jnp.float32)
    c_ref[...] = jnp.dot(h, w3t[...], preferred_element_type=jnp.float32) + b3[...]


def _pool_body(agg_ref, bm_ref, cm_ref, wdeg_ref, batch_ref,
               l1t, l1b, l2t, l2b, pred_ref, sums, counts):
    i = pl.program_id(0)
    agg = jnp.concatenate([agg_ref[0], agg_ref[1]], axis=1)
    h = jnp.maximum(agg - bm_ref[...] * wdeg_ref[...] + cm_ref[...], 0.0)
    bidx = batch_ref[...][:, 0]
    oh = (bidx[:, None] == lax.broadcasted_iota(jnp.int32, (R, G), 1)
          ).astype(jnp.float32)

    @pl.when(i == 0)
    def _():
        sums[...] = jnp.zeros((G, H), jnp.float32)
        counts[...] = jnp.zeros((G, 1), jnp.float32)

    sums[...] += lax.dot_general(oh, h, (((0,), (0,)), ((), ())),
                                 preferred_element_type=jnp.float32)
    counts[...] += jnp.sum(oh, axis=0)[:, None]

    @pl.when(i == GRID - 1)
    def _():
        gx = sums[...] / jnp.maximum(counts[...], 1.0)
        p = jnp.maximum(
            jnp.dot(gx, l1t[...], preferred_element_type=jnp.float32) + l1b[...],
            0.0)
        pred_ref[...] = jnp.dot(p, l2t[...],
                                preferred_element_type=jnp.float32) + l2b[...]


def _row_spec(cols):
    return pl.BlockSpec((R, cols), lambda i: (i, 0))


def _split_spec(cols):
    return pl.BlockSpec((2, R, cols), lambda i: (0, i, 0))


def _prep(x8, nwt, nb, w1t, b1, w2t, w3t, b3):
    return pl.pallas_call(
        _prep_body,
        grid=(GRID,),
        in_specs=[
            _row_spec(8), _full((8, H)), _full((1, H)),
            _full((H, H)), _full((1, H)), _full((H, H)),
            _full((H, H)), _full((1, H)),
        ],
        out_specs=[_split_spec(HH), _row_spec(H), _row_spec(H)],
        out_shape=[
            jax.ShapeDtypeStruct((2, N, HH), jnp.float32),
            jax.ShapeDtypeStruct((N, H), jnp.float32),
            jax.ShapeDtypeStruct((N, H), jnp.float32),
        ],
    )(x8, nwt, nb, w1t, b1, w2t, w3t, b3)


def _combine(agg, bm, cm, wdeg, w1t, b1, w2t, w3t, b3):
    return pl.pallas_call(
        _combine_body,
        grid=(GRID,),
        in_specs=[
            _split_spec(HH), _row_spec(H), _row_spec(H), _row_spec(1),
            _full((H, H)), _full((1, H)), _full((H, H)),
            _full((H, H)), _full((1, H)),
        ],
        out_specs=[_split_spec(HH), _row_spec(H), _row_spec(H)],
        out_shape=[
            jax.ShapeDtypeStruct((2, N, HH), jnp.float32),
            jax.ShapeDtypeStruct((N, H), jnp.float32),
            jax.ShapeDtypeStruct((N, H), jnp.float32),
        ],
    )(agg, bm, cm, wdeg, w1t, b1, w2t, w3t, b3)


def _pool(agg, bm, cm, wdeg, batch2, l1t, l1b, l2t, l2b):
    return pl.pallas_call(
        _pool_body,
        grid=(GRID,),
        in_specs=[
            _split_spec(HH), _row_spec(H), _row_spec(H), _row_spec(1),
            _row_spec(1),
            _full((H, G)), _full((1, G)), _full((G, 3)), _full((1, 3)),
        ],
        out_specs=[pl.BlockSpec((G, 3), lambda i: (0, 0))],
        out_shape=[jax.ShapeDtypeStruct((G, 3), jnp.float32)],
        scratch_shapes=[
            pltpu.VMEM((G, H), jnp.float32),
            pltpu.VMEM((G, 1), jnp.float32),
        ],
    )(agg, bm, cm, wdeg, batch2, l1t, l1b, l2t, l2b)[0]


def kernel(x, edge_index, edge_attr, batch, node_W, node_b,
           W1s, b1s, W2s, W3s, b3s, lin1_W, lin1_b, lin2_W, lin2_b):
    pad = EPAD - E
    src = jnp.concatenate([edge_index[0], jnp.zeros((pad,), jnp.int32)])
    dst = jnp.concatenate([edge_index[1], jnp.zeros((pad,), jnp.int32)])
    ew = jnp.concatenate([edge_attr, jnp.zeros((pad,), jnp.float32)])
    epack = jnp.concatenate(
        [src.reshape(-1, 1, CH), dst.reshape(-1, 1, CH),
         lax.bitcast_convert_type(ew, jnp.int32).reshape(-1, 1, CH)],
        axis=1)

    zz = jnp.zeros((ZR, HH), jnp.float32)
    x8 = jnp.concatenate([x, jnp.zeros((N, 3), jnp.float32)], axis=1)
    nwt = jnp.concatenate([node_W.T, jnp.zeros((3, H), jnp.float32)], axis=0)

    wparts = _wdeg_kernel(epack)
    wdeg = _wreduce(wparts)

    asp, bm, cm = _prep(
        x8, nwt, node_b[None, :],
        W1s[0].T, b1s[0][None, :], W2s[0].T, W3s[0].T, b3s[0][None, :])

    for i in (1, 2):
        agg = _edge_kernel(asp, epack, zz)
        asp, bm, cm = _combine(
            agg, bm, cm, wdeg,
            W1s[i].T, b1s[i][None, :], W2s[i].T, W3s[i].T, b3s[i][None, :])

    agg = _edge_kernel(asp, epack, zz)
    return _pool(agg, bm, cm, wdeg, batch[:, None],
                 lin1_W.T, lin1_b[None, :], lin2_W.T, lin2_b[None, :])


# speedup vs baseline: 1.0360x; 1.0312x over previous
import functools

import jax
import jax.numpy as jnp
from jax import lax
from jax.experimental import pallas as pl
from jax.experimental.pallas import tpu as pltpu
from jax.experimental.pallas import tpu_sc as plsc

N = 50000
E = 800000
G = 128
H = 64
HH = 32

NSUB = 16
EPAD = 819200
EPC = EPAD // NSUB
CH = 128
NCH = EPC // CH

ROWS_A = 3128
ROWS_B = 3080
ZR = 280

R = 2000
GRID = N // R


def _sc_mesh():
    return plsc.VectorSubcoreMesh(core_axis_name="c", subcore_axis_name="s")


@functools.partial(
    pl.kernel,
    mesh=_sc_mesh(),
    compiler_params=pltpu.CompilerParams(use_tc_tiling_on_sc=False, needs_layout_passes=False),
    out_type=jax.ShapeDtypeStruct((2, N, HH), jnp.float32),
    scratch_types=[
        pltpu.VMEM((10, 3, CH), jnp.int32),
        pltpu.VMEM((5, CH, HH), jnp.float32),
        pltpu.VMEM_SHARED((N, HH), jnp.float32),
        pltpu.SemaphoreType.DMA, pltpu.SemaphoreType.DMA,
        pltpu.SemaphoreType.DMA, pltpu.SemaphoreType.DMA,
        pltpu.SemaphoreType.DMA, pltpu.SemaphoreType.DMA,
        pltpu.SemaphoreType.DMA, pltpu.SemaphoreType.DMA,
        pltpu.SemaphoreType.DMA, pltpu.SemaphoreType.DMA,
        pltpu.SemaphoreType.DMA, pltpu.SemaphoreType.DMA,
        pltpu.SemaphoreType.DMA, pltpu.SemaphoreType.DMA,
        pltpu.SemaphoreType.DMA, pltpu.SemaphoreType.DMA,
        pltpu.SemaphoreType.DMA, pltpu.SemaphoreType.DMA,
        pltpu.SemaphoreType.DMA, pltpu.SemaphoreType.DMA,
    ],
)
def _edge_kernel(asp, ep_h, zz_h, out_h, ebuf, rows, acc,
                 es0, es1, es2, es3, es4, es5, es6, es7, es8, es9,
                 gs0, gs1, gs2, gs3, gs4, ss0, ss1, ss2, ss3, ss4):
    cid = lax.axis_index("c")
    sid = lax.axis_index("s")
    esems = [es0, es1, es2, es3, es4, es5, es6, es7, es8, es9]
    gsems = [gs0, gs1, gs2, gs3, gs4]
    ssems = [ss0, ss1, ss2, ss3, ss4]

    rbase = sid * ROWS_A

    def zcopy(j, _):
        pltpu.sync_copy(zz_h, acc.at[pl.ds(rbase + j * ZR, ZR)])
        return 0
    lax.fori_loop(0, ROWS_B // ZR, zcopy, 0)

    @pl.when(sid < NSUB - 1)
    def _():
        pltpu.sync_copy(zz_h.at[pl.ds(0, ROWS_A - ROWS_B)],
                        acc.at[pl.ds(rbase + ROWS_B, ROWS_A - ROWS_B)])

    plsc.subcore_barrier()

    gch0 = sid * NCH

    def e_issue(gj, eb):
        pltpu.async_copy(ep_h.at[gj], ebuf.at[eb], esems[eb])

    def e_wait(gj, eb):
        pltpu.make_async_copy(ep_h.at[gj], ebuf.at[eb], esems[eb]).wait()

    def g_issue(eb, b):
        pltpu.async_copy(asp.at[cid].at[ebuf.at[eb].at[0]], rows.at[b],
                         gsems[b])

    def g_wait(eb, b):
        pltpu.make_async_copy(asp.at[cid].at[ebuf.at[eb].at[0]], rows.at[b],
                              gsems[b]).wait()

    def s_issue(eb, b):
        pltpu.async_copy(rows.at[b], acc.at[ebuf.at[eb].at[1]], ssems[b],
                         add=True)

    def s_wait(eb, b):
        pltpu.make_async_copy(rows.at[b], acc.at[ebuf.at[eb].at[1]],
                              ssems[b]).wait()

    def scale(eb, b):
        def body(g, _):
            wi = ebuf[eb, 2, pl.ds(g * 16, 16)]
            w = plsc.bitcast(wi, jnp.float32)
            base = g * 16
            for l in range(16):
                s = w[l]
                e = base + l
                rows[b, e, pl.ds(0, 16)] = rows[b, e, pl.ds(0, 16)] * s
                rows[b, e, pl.ds(16, 16)] = rows[b, e, pl.ds(16, 16)] * s
            return 0
        lax.fori_loop(0, CH // 16, body, 0)

    for eb in range(6):
        e_issue(gch0 + eb, eb)
    for k in range(3):
        e_wait(gch0 + k, k)
        g_issue(k, k)

    NOUT = NCH // 10

    def outer(jj, _):
        for bb in range(10):
            j = 10 * jj + bb
            b = bb % 5
            bp = (bb + 3) % 5
            ebp = (bb + 8) % 10
            eb6 = (bb + 6) % 10
            eb3 = (bb + 3) % 10

            g_wait(bb, b)

            if bb >= 2:
                s_wait(ebp, bp)
            else:
                @pl.when(jj >= 1)
                def _():
                    s_wait(ebp, bp)

            if bb < 4:
                e_issue(gch0 + j + 6, eb6)
            else:
                @pl.when(jj < NOUT - 1)
                def _():
                    e_issue(gch0 + j + 6, eb6)

            if bb < 7:
                e_wait(gch0 + j + 3, eb3)
                g_issue(eb3, bp)
            else:
                @pl.when(jj < NOUT - 1)
                def _():
                    e_wait(gch0 + j + 3, eb3)
                    g_issue(eb3, bp)

            scale(bb, b)
            s_issue(bb, b)
        return 0
    lax.fori_loop(0, NOUT, outer, 0)

    s_wait(8, 3)
    s_wait(9, 4)

    plsc.subcore_barrier()

    pltpu.sync_copy(acc.at[pl.ds(rbase, ROWS_B)],
                    out_h.at[cid].at[pl.ds(rbase, ROWS_B)])

    @pl.when(sid < NSUB - 1)
    def _():
        pltpu.sync_copy(acc.at[pl.ds(rbase + ROWS_B, ROWS_A - ROWS_B)],
                        out_h.at[cid].at[pl.ds(rbase + ROWS_B, ROWS_A - ROWS_B)])


NW = 32
EPT = EPAD // NW
NCHW = EPT // CH


@functools.partial(
    pl.kernel,
    mesh=_sc_mesh(),
    compiler_params=pltpu.CompilerParams(use_tc_tiling_on_sc=False, needs_layout_passes=False),
    out_type=jax.ShapeDtypeStruct((NW, N), jnp.float32),
    scratch_types=[
        pltpu.VMEM((8, 3, CH), jnp.int32),
        pltpu.VMEM((N,), jnp.float32),
        pltpu.SemaphoreType.DMA, pltpu.SemaphoreType.DMA,
        pltpu.SemaphoreType.DMA, pltpu.SemaphoreType.DMA,
        pltpu.SemaphoreType.DMA, pltpu.SemaphoreType.DMA,
        pltpu.SemaphoreType.DMA, pltpu.SemaphoreType.DMA,
    ],
)
def _wdeg_kernel(ep_h, out_h, ebuf, accv,
                 es0, es1, es2, es3, es4, es5, es6, es7):
    cid = lax.axis_index("c")
    sid = lax.axis_index("s")
    wid = sid * 2 + cid
    esems = [es0, es1, es2, es3, es4, es5, es6, es7]

    zvec = jnp.zeros((16,), jnp.float32)

    def zfill(r, _):
        accv[pl.ds(r * 16, 16)] = zvec
        return 0
    lax.fori_loop(0, N // 16, zfill, 0)

    gch0 = wid * NCHW

    def e_issue(gj, eb):
        pltpu.async_copy(ep_h.at[gj], ebuf.at[eb], esems[eb])

    def e_wait(gj, eb):
        pltpu.make_async_copy(ep_h.at[gj], ebuf.at[eb], esems[eb]).wait()

    for eb in range(6):
        e_issue(gch0 + eb, eb)

    def outer(jj, _):
        for bb in range(8):
            j = 8 * jj + bb
            e_wait(gch0 + j, bb)

            if bb < 2:
                e_issue(gch0 + j + 6, (bb + 6) % 8)
            else:
                @pl.when(jj < NCHW // 8 - 1)
                def _():
                    e_issue(gch0 + j + 6, (bb + 6) % 8)

            def fill(g, _):
                idx = ebuf[bb, 1, pl.ds(g * 16, 16)]
                w = plsc.bitcast(ebuf[bb, 2, pl.ds(g * 16, 16)], jnp.float32)
                plsc.addupdate_scatter(accv, [idx], w)
                return 0
            lax.fori_loop(0, CH // 16, fill, 0)
        return 0
    lax.fori_loop(0, NCHW // 8, outer, 0)

    pltpu.sync_copy(accv, out_h.at[wid])


def _full(shape):
    return pl.BlockSpec(shape, lambda i: tuple(0 for _ in shape))


def _prep_body(x_ref, nwt, nb, w1t, b1, w2t, w3t, b3,
               asp_ref, b_ref, c_ref):
    h = jnp.dot(x_ref[...], nwt[...], preferred_element_type=jnp.float32) + nb[...]
    a = jnp.dot(h, w1t[...], preferred_element_type=jnp.float32) + b1[...]
    asp_ref[0] = a[:, :HH]
    asp_ref[1] = a[:, HH:]
    b_ref[...] = jnp.dot(h, w2t[...], preferred_element_type=jnp.float32)
    c_ref[...] = jnp.dot(h, w3t[...], preferred_element_type=jnp.float32) + b3[...]


def _combine_body(agg_ref, bm_ref, cm_ref, wp_ref, w1t, b1, w2t, w3t, b3,
                  asp_ref, b_ref, c_ref):
    agg = jnp.concatenate([agg_ref[0], agg_ref[1]], axis=1)
    wdeg = jnp.sum(wp_ref[...], axis=1)[:, None]
    h = jnp.maximum(agg - bm_ref[...] * wdeg + cm_ref[...], 0.0)
    a = jnp.dot(h, w1t[...], preferred_element_type=jnp.float32) + b1[...]
    asp_ref[0] = a[:, :HH]
    asp_ref[1] = a[:, HH:]
    b_ref[...] = jnp.dot(h, w2t[...], preferred_element_type=jnp.float32)
    c_ref[...] = jnp.dot(h, w3t[...], preferred_element_type=jnp.float32) + b3[...]


def _pool_body(agg_ref, bm_ref, cm_ref, wp_ref, batch_ref,
               l1t, l1b, l2t, l2b, pred_ref, sums, counts):
    i = pl.program_id(0)
    agg = jnp.concatenate([agg_ref[0], agg_ref[1]], axis=1)
    wdeg = jnp.sum(wp_ref[...], axis=1)[:, None]
    h = jnp.maximum(agg - bm_ref[...] * wdeg + cm_ref[...], 0.0)
    bidx = batch_ref[...][:, 0]
    oh = (bidx[:, None] == lax.broadcasted_iota(jnp.int32, (R, G), 1)
          ).astype(jnp.float32)

    @pl.when(i == 0)
    def _():
        sums[...] = jnp.zeros((G, H), jnp.float32)
        counts[...] = jnp.zeros((G, 1), jnp.float32)

    sums[...] += lax.dot_general(oh, h, (((0,), (0,)), ((), ())),
                                 preferred_element_type=jnp.float32)
    counts[...] += jnp.sum(oh, axis=0)[:, None]

    @pl.when(i == GRID - 1)
    def _():
        gx = sums[...] / jnp.maximum(counts[...], 1.0)
        p = jnp.maximum(
            jnp.dot(gx, l1t[...], preferred_element_type=jnp.float32) + l1b[...],
            0.0)
        pred_ref[...] = jnp.dot(p, l2t[...],
                                preferred_element_type=jnp.float32) + l2b[...]


def _row_spec(cols):
    return pl.BlockSpec((R, cols), lambda i: (i, 0))


def _split_spec(cols):
    return pl.BlockSpec((2, R, cols), lambda i: (0, i, 0))


def _prep(x8, nwt, nb, w1t, b1, w2t, w3t, b3):
    return pl.pallas_call(
        _prep_body,
        grid=(GRID,),
        in_specs=[
            _row_spec(8), _full((8, H)), _full((1, H)),
            _full((H, H)), _full((1, H)), _full((H, H)),
            _full((H, H)), _full((1, H)),
        ],
        out_specs=[_split_spec(HH), _row_spec(H), _row_spec(H)],
        out_shape=[
            jax.ShapeDtypeStruct((2, N, HH), jnp.float32),
            jax.ShapeDtypeStruct((N, H), jnp.float32),
            jax.ShapeDtypeStruct((N, H), jnp.float32),
        ],
    )(x8, nwt, nb, w1t, b1, w2t, w3t, b3)


def _combine(agg, bm, cm, wdeg, w1t, b1, w2t, w3t, b3):
    return pl.pallas_call(
        _combine_body,
        grid=(GRID,),
        in_specs=[
            _split_spec(HH), _row_spec(H), _row_spec(H), _row_spec(NW),
            _full((H, H)), _full((1, H)), _full((H, H)),
            _full((H, H)), _full((1, H)),
        ],
        out_specs=[_split_spec(HH), _row_spec(H), _row_spec(H)],
        out_shape=[
            jax.ShapeDtypeStruct((2, N, HH), jnp.float32),
            jax.ShapeDtypeStruct((N, H), jnp.float32),
            jax.ShapeDtypeStruct((N, H), jnp.float32),
        ],
    )(agg, bm, cm, wdeg, w1t, b1, w2t, w3t, b3)


def _pool(agg, bm, cm, wdeg, batch2, l1t, l1b, l2t, l2b):
    return pl.pallas_call(
        _pool_body,
        grid=(GRID,),
        in_specs=[
            _split_spec(HH), _row_spec(H), _row_spec(H), _row_spec(NW),
            _row_spec(1),
            _full((H, G)), _full((1, G)), _full((G, 3)), _full((1, 3)),
        ],
        out_specs=[pl.BlockSpec((G, 3), lambda i: (0, 0))],
        out_shape=[jax.ShapeDtypeStruct((G, 3), jnp.float32)],
        scratch_shapes=[
            pltpu.VMEM((G, H), jnp.float32),
            pltpu.VMEM((G, 1), jnp.float32),
        ],
    )(agg, bm, cm, wdeg, batch2, l1t, l1b, l2t, l2b)[0]


def kernel(x, edge_index, edge_attr, batch, node_W, node_b,
           W1s, b1s, W2s, W3s, b3s, lin1_W, lin1_b, lin2_W, lin2_b):
    pad = EPAD - E
    src = jnp.concatenate([edge_index[0], jnp.zeros((pad,), jnp.int32)])
    dst = jnp.concatenate([edge_index[1], jnp.zeros((pad,), jnp.int32)])
    ew = jnp.concatenate([edge_attr, jnp.zeros((pad,), jnp.float32)])
    epack = jnp.concatenate(
        [src.reshape(-1, 1, CH), dst.reshape(-1, 1, CH),
         lax.bitcast_convert_type(ew, jnp.int32).reshape(-1, 1, CH)],
        axis=1)

    zz = jnp.zeros((ZR, HH), jnp.float32)
    x8 = jnp.concatenate([x, jnp.zeros((N, 3), jnp.float32)], axis=1)
    nwt = jnp.concatenate([node_W.T, jnp.zeros((3, H), jnp.float32)], axis=0)

    wpt = _wdeg_kernel(epack).T

    asp, bm, cm = _prep(
        x8, nwt, node_b[None, :],
        W1s[0].T, b1s[0][None, :], W2s[0].T, W3s[0].T, b3s[0][None, :])

    for i in (1, 2):
        agg = _edge_kernel(asp, epack, zz)
        asp, bm, cm = _combine(
            agg, bm, cm, wpt,
            W1s[i].T, b1s[i][None, :], W2s[i].T, W3s[i].T, b3s[i][None, :])

    agg = _edge_kernel(asp, epack, zz)
    return _pool(agg, bm, cm, wpt, batch[:, None],
                 lin1_W.T, lin1_b[None, :], lin2_W.T, lin2_b[None, :])
